# baseline (speedup 1.0000x reference)
import functools

import jax
import jax.numpy as jnp
from jax import lax
from jax.experimental import pallas as pl
from jax.experimental.pallas import tpu as pltpu
from jax.experimental.pallas import tpu_sc as plsc

G = 16
N2 = 32
T = G * G
C = 3
N2K = C * N2 * N2
B = 64

NC, NS = 2, 16
NW = NC * NS
ROWS_PER_W = (B * T) // NW
CHUNKS = ROWS_PER_W // G
MAXFIX = 256
QF = N2K // 128


def _sc_body(x_hbm, out_hbm, rowbuf, gat_sem, scat_sem):
    wid = lax.axis_index("s") * NC + lax.axis_index("c")

    def gather_chunk(g, slot):
        band = wid * CHUNKS + g
        b = lax.div(band, G)
        g1 = lax.rem(band, G)

        def row_body(i, carry):
            tg_l = lax.div(i, 8)
            s = lax.rem(i, 8)
            co = lax.div(i, 4)
            lq = lax.rem(i, 4)
            for c in range(C):
                for rb in range(4):
                    ro = g1 * 4 + rb
                    q0 = c * 8 + rb * 2
                    pltpu.make_async_copy(
                        x_hbm.at[b, c, ro, co, :, :, lq, :],
                        rowbuf.at[slot, tg_l, pl.ds(q0, 2), s, :, :],
                        gat_sem.at[slot],
                    ).start()
            return carry

        lax.fori_loop(0, G, row_body, 0)

    def wait_gathers(slot):
        pltpu.make_async_copy(
            out_hbm.at[pl.ds(0, 2)],
            rowbuf.at[slot],
            gat_sem.at[slot],
        ).wait()

    def scatter_chunk(h, slot):
        band = wid * CHUNKS + h
        pltpu.async_copy(
            rowbuf.at[slot],
            out_hbm.at[pl.ds(band * 2, 2)],
            scat_sem.at[slot],
        )

    def wait_scatter(slot):
        pltpu.make_async_copy(
            out_hbm.at[pl.ds(0, 2)],
            rowbuf.at[slot],
            scat_sem.at[slot],
        ).wait()

    def loop_body(g, carry):
        slot = lax.rem(g, 2)

        @pl.when(g < CHUNKS)
        def _issue():
            @pl.when(g >= 2)
            def _reuse():
                wait_scatter(slot)

            gather_chunk(g, slot)

        @pl.when(g >= 1)
        def _process():
            h = g - 1
            sloth = lax.rem(h, 2)
            wait_gathers(sloth)
            scatter_chunk(h, sloth)

        return carry

    lax.fori_loop(0, CHUNKS + 1, loop_body, 0)
    wait_scatter(0)
    wait_scatter(1)


def _fix_kernel(rows_ref, cnt_ref, repl_ref, _, out_ref, sem):
    cnt = cnt_ref[0, 0]

    def start(k, carry):
        pltpu.make_async_copy(
            repl_ref, out_ref.at[pl.ds(rows_ref[0, k], 1), :], sem
        ).start()
        return carry

    def drain(k, carry):
        pltpu.make_async_copy(
            repl_ref, out_ref.at[pl.ds(0, 1), :], sem
        ).wait()
        return carry

    lax.fori_loop(0, cnt, start, 0)
    lax.fori_loop(0, cnt, drain, 0)


def kernel(X):
    b = X.shape[0]
    k1, k2 = jax.random.split(jax.random.key(1))
    idx = jax.random.bernoulli(k1, 1.0 / T, (b * T,))
    repl = jnp.tanh(jax.random.normal(k2, (N2K,), dtype=jnp.float32))

    xl = X.reshape(b, C, 64, 2, 4, 4, 4, 32).transpose(0, 1, 2, 5, 3, 4, 6, 7)

    mesh = plsc.VectorSubcoreMesh(
        core_axis_name="c", subcore_axis_name="s",
        num_cores=NC, num_subcores=NS,
    )
    sc_fn = functools.partial(
        pl.kernel,
        out_type=jax.ShapeDtypeStruct((b * T // 8, QF, 8, 4, 32), jnp.float32),
        mesh=mesh,
        scratch_types=[
            pltpu.VMEM((2, 2, QF, 8, 4, 32), jnp.float32),
            pltpu.SemaphoreType.DMA((2,)),
            pltpu.SemaphoreType.DMA((2,)),
        ],
        compiler_params=pltpu.CompilerParams(use_tc_tiling_on_sc=False),
    )(_sc_body)

    outl = sc_fn(xl)
    patched = (outl.reshape(b, T // 8, QF, 8, 128)
               .transpose(0, 1, 3, 2, 4)
               .reshape(b * T, N2K))

    rows = jnp.nonzero(idx, size=MAXFIX, fill_value=0)[0]
    rows2 = rows.astype(jnp.int32).reshape(1, MAXFIX)
    cnt2 = jnp.sum(idx).astype(jnp.int32).reshape(1, 1)

    out = pl.pallas_call(
        _fix_kernel,
        in_specs=[
            pl.BlockSpec(memory_space=pltpu.MemorySpace.SMEM),
            pl.BlockSpec(memory_space=pltpu.MemorySpace.SMEM),
            pl.BlockSpec(memory_space=pltpu.MemorySpace.VMEM),
            pl.BlockSpec(memory_space=pl.ANY),
        ],
        out_specs=pl.BlockSpec(memory_space=pl.ANY),
        out_shape=jax.ShapeDtypeStruct((b * T, N2K), jnp.float32),
        scratch_shapes=[pltpu.SemaphoreType.DMA],
        input_output_aliases={3: 0},
    )(rows2, cnt2, repl.reshape(1, N2K), patched)

    return out.reshape(b, T, N2K), idx


# speedup vs baseline: 2.5922x; 2.5922x over previous
import jax
import jax.numpy as jnp
from jax.experimental import pallas as pl

G = 16
N2 = 32
T = G * G
C = 3
N2K = C * N2 * N2


def _patch_kernel(x_ref, m_ref, repl_ref, out_ref):
    nb = x_ref.shape[2] // N2
    x = x_ref[0]
    y = x.reshape(C, nb, N2, G, N2).transpose(1, 3, 0, 2, 4).reshape(nb * G, N2K)
    m = m_ref[0, 0, 0, :]
    repl = repl_ref[0]
    out_ref[0] = jnp.where(m[:, None] > 0.5, repl[None, :], y)


def kernel(X):
    b = X.shape[0]
    k1, k2 = jax.random.split(jax.random.key(1))
    idx = jax.random.bernoulli(k1, 1.0 / T, (b * T,))
    repl = jnp.tanh(jax.random.normal(k2, (N2K,), dtype=jnp.float32))

    NB = 8
    m4 = idx.reshape(b, G // NB, 1, NB * G).astype(jnp.float32)
    repl2 = repl.reshape(1, N2K)

    out = pl.pallas_call(
        _patch_kernel,
        grid=(b, G // NB),
        in_specs=[
            pl.BlockSpec((1, C, NB * N2, G * N2), lambda i, j: (i, 0, j, 0)),
            pl.BlockSpec((1, 1, 1, NB * G), lambda i, j: (i, j, 0, 0)),
            pl.BlockSpec((1, N2K), lambda i, j: (0, 0)),
        ],
        out_specs=pl.BlockSpec((1, NB * G, N2K), lambda i, j: (i, j, 0)),
        out_shape=jax.ShapeDtypeStruct((b, T, N2K), jnp.float32),
    )(X, m4, repl2)

    return out, idx
